# baseline (speedup 1.0000x reference)
import functools

import jax
import jax.numpy as jnp
from jax import lax
from jax.experimental import pallas as pl
from jax.experimental.pallas import tpu as pltpu
from jax.experimental.pallas import tpu_sc as plsc

NC = 2
NS = 16
NW = NC * NS
LANES = 16

GATHER_IDX = 100
GATHERS = 4
CHUNK = GATHER_IDX * GATHERS


@functools.partial(jax.jit, static_argnums=(3, 4, 5))
def _fused_embed(xf, table, pe2, nchunk, seq, emb_dim):
    mesh = plsc.VectorSubcoreMesh(core_axis_name="c", subcore_axis_name="s")

    @functools.partial(
        pl.kernel,
        out_type=jax.ShapeDtypeStruct((NW * nchunk * CHUNK, emb_dim), jnp.float32),
        mesh=mesh,
        scratch_types=[
            pltpu.VMEM((GATHERS, GATHER_IDX), jnp.int32),
            pltpu.VMEM((CHUNK, emb_dim), jnp.float32),
            pltpu.VMEM((CHUNK, emb_dim), jnp.float32),
            pltpu.SemaphoreType.DMA,
        ],
        compiler_params=pltpu.CompilerParams(use_tc_tiling_on_sc=False),
    )
    def body(idx_hbm, table_hbm, pe_hbm, out_hbm, idx_v, rows_v, pe_v, sem):
        wid = lax.axis_index("s") * NC + lax.axis_index("c")
        pltpu.sync_copy(pe_hbm, pe_v)

        def chunk_body(c, carry):
            pltpu.sync_copy(idx_hbm.at[wid, c], idx_v)
            cps = [
                pltpu.async_copy(
                    table_hbm.at[idx_v.at[j]],
                    rows_v.at[pl.ds(j * GATHER_IDX, GATHER_IDX)],
                    sem,
                )
                for j in range(GATHERS)
            ]
            for cp in cps:
                cp.wait()

            def row_body(r, rcarry):
                for dseg in range(emb_dim // LANES):
                    sl = pl.ds(dseg * LANES, LANES)
                    rows_v[r, sl] = rows_v[r, sl] + pe_v[r, sl]
                return rcarry

            lax.fori_loop(0, CHUNK, row_body, 0, unroll=4)
            base = (wid * nchunk + c) * CHUNK
            pltpu.sync_copy(rows_v, out_hbm.at[pl.ds(base, CHUNK)])
            return carry

        lax.fori_loop(0, nchunk, chunk_body, 0)

    return body(xf, table, pe2)


def kernel(x, table, pe):
    batch, seq = x.shape
    emb_dim = table.shape[1]
    total = batch * seq
    assert total % (NW * CHUNK) == 0 and seq * 2 == CHUNK
    nchunk = total // (NW * CHUNK)
    xf = x.reshape(NW, nchunk, GATHERS, GATHER_IDX)
    pe2 = jnp.concatenate([pe[:seq], pe[:seq]], axis=0)
    out = _fused_embed(xf, table, pe2, nchunk, seq, emb_dim)
    return out.reshape(batch, seq, emb_dim)
